# baseline (speedup 1.0000x reference)
import functools

import jax
import jax.numpy as jnp
from jax import lax
from jax.experimental import pallas as pl
from jax.experimental.pallas import tpu as pltpu
from jax.experimental.pallas import tpu_sc as plsc

EMB = 64
VOCAB = 1000000
NUM_CORES = 2
NUM_SUBCORES = 16
NUM_WORKERS = NUM_CORES * NUM_SUBCORES

N_FULL_IB = VOCAB // 128
TAIL = VOCAB - N_FULL_IB * 128
IB_PER_W = 246
DUMMY_ROW = 1000064
ROWS_PAD = DUMMY_ROW + 128

_MESH = dict(core_axis_name="c", subcore_axis_name="s")


def _worker_id():
    return lax.axis_index("s") * NUM_CORES + lax.axis_index("c")


def _iota16(base):
    return lax.iota(jnp.int32, 16) + base


def _transpose16(src, dst, temp, rows_grp, cols_grp):
    iota1 = lax.iota(jnp.int32, 16)
    iota17 = iota1 * 17

    @plsc.parallel_loop(0, rows_grp * cols_grp, step=1, unroll=2)
    def _(blk):
        a = blk // cols_grp
        c = lax.rem(blk, cols_grp)
        base = blk * 288
        vals = []
        for q in range(16):
            vals.append(src[a * 16 + q, pl.ds(c * 16, 16)])
        for q in range(16):
            plsc.store_scatter(temp, [iota1 + (base + q * 17)], vals[q])
        loaded = []
        for p in range(16):
            loaded.append(plsc.load_gather(temp, [iota17 + (base + p)]))
        for p in range(16):
            dst[c * 16 + p, pl.ds(a * 16, 16)] = loaded[p]


@jax.jit
def _rowize(tT, tail):
    @functools.partial(
        pl.kernel,
        out_type=jax.ShapeDtypeStruct((ROWS_PAD, 128), jnp.float32),
        mesh=plsc.VectorSubcoreMesh(**_MESH),
        scratch_types=[
            pltpu.VMEM((2, EMB, 128), jnp.float32),
            pltpu.VMEM((2, 128, 128), jnp.float32),
            pltpu.VMEM((32 * 288,), jnp.float32),
            pltpu.SemaphoreType.DMA,
            pltpu.SemaphoreType.DMA,
            pltpu.SemaphoreType.DMA,
            pltpu.SemaphoreType.DMA,
        ],
        compiler_params=pltpu.CompilerParams(use_tc_tiling_on_sc=True, needs_layout_passes=False),
    )
    def k(tT_hbm, tail_hbm, rows_hbm, stage_v, trows_v, temp_v, g0, g1, s0, s1):
        wid = _worker_id()
        ib_lo = wid * IB_PER_W
        gsems = (g0, g1)
        ssems = (s0, s1)

        def fire_load(t, b):
            ib = lax.min(ib_lo + t, N_FULL_IB - 1)
            col0 = pl.multiple_of(ib * 128, 128)
            pltpu.async_copy(
                tT_hbm.at[pl.ds(0, EMB), pl.ds(col0, 128)],
                stage_v.at[b], gsems[b],
            )

        def wait_load(b):
            pltpu.make_async_copy(
                tT_hbm.at[pl.ds(0, EMB), pl.ds(0, 128)], stage_v.at[b],
                gsems[b],
            ).wait()

        def fire_store(t, b):
            ib = ib_lo + t
            row0 = pl.multiple_of(
                jnp.where(ib < N_FULL_IB, ib * 128, DUMMY_ROW), 128
            )
            pltpu.async_copy(
                trows_v.at[b],
                rows_hbm.at[pl.ds(row0, 128), pl.ds(0, 128)],
                ssems[b],
            )

        def wait_store(b):
            pltpu.make_async_copy(
                trows_v.at[b],
                rows_hbm.at[pl.ds(0, 128), pl.ds(0, 128)], ssems[b],
            ).wait()

        def transpose(b):
            _transpose16(stage_v.at[b], trows_v.at[b], temp_v,
                         EMB // 16, 128 // 16)

        fire_load(0, 0)

        def pair(p, carry):
            for b in range(2):
                t = 2 * p + b

                @pl.when(t + 1 < IB_PER_W)
                def _():
                    fire_load(t + 1, 1 - b)

                wait_load(b)

                @pl.when(t >= 2)
                def _():
                    wait_store(b)

                transpose(b)
                fire_store(t, b)
            return carry

        lax.fori_loop(0, IB_PER_W // 2, pair, 0)
        wait_store(0)
        wait_store(1)

        @pl.when(wid == NUM_WORKERS - 1)
        def _():
            pltpu.sync_copy(
                tail_hbm,
                rows_hbm.at[pl.ds(N_FULL_IB * 128, TAIL), pl.ds(0, 128)],
            )

    return k(tT, tail)


@jax.jit
def _gather(xcol, rows):
    n_j = xcol.shape[0] // (NUM_WORKERS * 128)
    per_w = n_j * 128

    @functools.partial(
        pl.kernel,
        out_type=jax.ShapeDtypeStruct((n_j, 8, NUM_WORKERS, 8, 128), jnp.float32),
        mesh=plsc.VectorSubcoreMesh(**_MESH),
        scratch_types=[
            pltpu.VMEM((per_w,), jnp.int32),
            pltpu.VMEM((2, 128, 128), jnp.float32),
            pltpu.VMEM((2, EMB, 128), jnp.float32),
            pltpu.VMEM((32 * 288,), jnp.float32),
            pltpu.SemaphoreType.DMA,
            pltpu.SemaphoreType.DMA,
            pltpu.SemaphoreType.DMA,
            pltpu.SemaphoreType.DMA,
        ],
        compiler_params=pltpu.CompilerParams(use_tc_tiling_on_sc=True, needs_layout_passes=False),
    )
    def k(xcol_hbm, rows_hbm, out_hbm, idx_v, rows_v, tiles_v, temp_v, g0, g1, s0, s1):
        wid = _worker_id()
        gsems = (g0, g1)
        ssems = (s0, s1)

        base = pl.multiple_of(wid * per_w, 128)
        pltpu.sync_copy(xcol_hbm.at[pl.ds(base, per_w)], idx_v)

        def fire_gather(j, b):
            off = pl.multiple_of(j * 128, 128)
            pltpu.async_copy(
                rows_hbm.at[idx_v.at[pl.ds(off, 128)]], rows_v.at[b], gsems[b],
            )

        def wait_gather(b):
            pltpu.make_async_copy(
                rows_hbm.at[pl.ds(0, 128)], rows_v.at[b], gsems[b],
            ).wait()

        def fire_stores(j, b):
            for kb in range(8):
                pltpu.async_copy(
                    tiles_v.at[b, pl.ds(kb * 8, 8), :],
                    out_hbm.at[j, kb, wid], ssems[b],
                )

        def wait_stores(b):
            pltpu.make_async_copy(
                rows_hbm.at[pl.ds(0, EMB), pl.ds(0, 128)], tiles_v.at[b],
                ssems[b],
            ).wait()

        def transpose(b):
            _transpose16(rows_v.at[b], tiles_v.at[b], temp_v,
                         128 // 16, EMB // 16)

        fire_gather(0, 0)

        def pair(p, carry):
            for b in range(2):
                j = 2 * p + b

                @pl.when(j + 1 < n_j)
                def _():
                    fire_gather(j + 1, 1 - b)

                wait_gather(b)

                @pl.when(j >= 2)
                def _():
                    wait_stores(b)

                transpose(b)
                fire_stores(j, b)
            return carry

        lax.fori_loop(0, n_j // 2, pair, 0)
        wait_stores(0)
        wait_stores(1)

    return k(xcol, rows)


def kernel(x, table):
    n_i, n_j = x.shape
    xcol = (
        x.T.astype(jnp.int32)
        .reshape(n_j, NUM_WORKERS, 128)
        .transpose(1, 0, 2)
        .reshape(-1)
    )
    tT = table.T
    tail = jnp.pad(
        lax.slice(table, (N_FULL_IB * 128, 0), (VOCAB, EMB)),
        ((0, 0), (0, 128 - EMB)),
    )
    rows = _rowize(tT, tail)
    out5d = _gather(xcol, rows)
    return out5d.transpose(2, 4, 0, 1, 3).reshape(n_i, n_j, EMB)


# speedup vs baseline: 1.0123x; 1.0123x over previous
import functools

import jax
import jax.numpy as jnp
from jax import lax
from jax.experimental import pallas as pl
from jax.experimental.pallas import tpu as pltpu
from jax.experimental.pallas import tpu_sc as plsc

EMB = 64
VOCAB = 1000000
NUM_CORES = 2
NUM_SUBCORES = 16
NUM_WORKERS = NUM_CORES * NUM_SUBCORES

N_FULL_IB = VOCAB // 128
TAIL = VOCAB - N_FULL_IB * 128
IB_PER_W = 246
DUMMY_ROW = 1000064
ROWS_PAD = DUMMY_ROW + 128

_MESH = dict(core_axis_name="c", subcore_axis_name="s")


def _worker_id():
    return lax.axis_index("s") * NUM_CORES + lax.axis_index("c")


def _iota16(base):
    return lax.iota(jnp.int32, 16) + base


def _transpose16(src, dst, temp, rows_grp, cols_grp):
    iota1 = lax.iota(jnp.int32, 16)
    iota17 = iota1 * 17

    @plsc.parallel_loop(0, rows_grp * cols_grp, step=1, unroll=2)
    def _(blk):
        a = blk // cols_grp
        c = lax.rem(blk, cols_grp)
        base = blk * 288
        vals = []
        for q in range(16):
            vals.append(src[a * 16 + q, pl.ds(c * 16, 16)])
        for q in range(16):
            plsc.store_scatter(temp, [iota1 + (base + q * 17)], vals[q])
        loaded = []
        for p in range(16):
            loaded.append(plsc.load_gather(temp, [iota17 + (base + p)]))
        for p in range(16):
            dst[c * 16 + p, pl.ds(a * 16, 16)] = loaded[p]


@jax.jit
def _rowize(tT, tail):
    @functools.partial(
        pl.kernel,
        out_type=jax.ShapeDtypeStruct((ROWS_PAD, 128), jnp.float32),
        mesh=plsc.VectorSubcoreMesh(**_MESH),
        scratch_types=[
            pltpu.VMEM((3, EMB, 128), jnp.float32),
            pltpu.VMEM((3, 128, 128), jnp.float32),
            pltpu.VMEM((32 * 288,), jnp.float32),
            pltpu.SemaphoreType.DMA,
            pltpu.SemaphoreType.DMA,
            pltpu.SemaphoreType.DMA,
            pltpu.SemaphoreType.DMA,
            pltpu.SemaphoreType.DMA,
            pltpu.SemaphoreType.DMA,
        ],
        compiler_params=pltpu.CompilerParams(use_tc_tiling_on_sc=True, needs_layout_passes=False),
    )
    def k(tT_hbm, tail_hbm, rows_hbm, stage_v, trows_v, temp_v, g0, g1, g2, s0, s1, s2):
        wid = _worker_id()
        ib_lo = wid * IB_PER_W
        gsems = (g0, g1, g2)
        ssems = (s0, s1, s2)

        def fire_load(t, b):
            ib = lax.min(ib_lo + t, N_FULL_IB - 1)
            col0 = pl.multiple_of(ib * 128, 128)
            pltpu.async_copy(
                tT_hbm.at[pl.ds(0, EMB), pl.ds(col0, 128)],
                stage_v.at[b], gsems[b],
            )

        def wait_load(b):
            pltpu.make_async_copy(
                tT_hbm.at[pl.ds(0, EMB), pl.ds(0, 128)], stage_v.at[b],
                gsems[b],
            ).wait()

        def fire_store(t, b):
            ib = ib_lo + t
            row0 = pl.multiple_of(
                jnp.where(ib < N_FULL_IB, ib * 128, DUMMY_ROW), 128
            )
            pltpu.async_copy(
                trows_v.at[b],
                rows_hbm.at[pl.ds(row0, 128), pl.ds(0, 128)],
                ssems[b],
            )

        def wait_store(b):
            pltpu.make_async_copy(
                trows_v.at[b],
                rows_hbm.at[pl.ds(0, 128), pl.ds(0, 128)], ssems[b],
            ).wait()

        def transpose(b):
            _transpose16(stage_v.at[b], trows_v.at[b], temp_v,
                         EMB // 16, 128 // 16)

        fire_load(0, 0)
        fire_load(1, 1)

        def trio(p, carry):
            for b in range(3):
                t = 3 * p + b

                @pl.when(t + 2 < IB_PER_W)
                def _():
                    fire_load(t + 2, (b + 2) % 3)

                wait_load(b)

                @pl.when(t >= 3)
                def _():
                    wait_store(b)

                transpose(b)
                fire_store(t, b)
            return carry

        lax.fori_loop(0, IB_PER_W // 3, trio, 0)
        wait_store(0)
        wait_store(1)
        wait_store(2)

        @pl.when(wid == NUM_WORKERS - 1)
        def _():
            pltpu.sync_copy(
                tail_hbm,
                rows_hbm.at[pl.ds(N_FULL_IB * 128, TAIL), pl.ds(0, 128)],
            )

    return k(tT, tail)


@jax.jit
def _gather(xcol, rows):
    n_j = xcol.shape[0] // (NUM_WORKERS * 128)
    per_w = n_j * 128

    @functools.partial(
        pl.kernel,
        out_type=jax.ShapeDtypeStruct((n_j, 8, NUM_WORKERS, 8, 128), jnp.float32),
        mesh=plsc.VectorSubcoreMesh(**_MESH),
        scratch_types=[
            pltpu.VMEM((per_w,), jnp.int32),
            pltpu.VMEM((2, 128, 128), jnp.float32),
            pltpu.VMEM((2, EMB, 128), jnp.float32),
            pltpu.VMEM((32 * 288,), jnp.float32),
            pltpu.SemaphoreType.DMA,
            pltpu.SemaphoreType.DMA,
            pltpu.SemaphoreType.DMA,
            pltpu.SemaphoreType.DMA,
        ],
        compiler_params=pltpu.CompilerParams(use_tc_tiling_on_sc=True, needs_layout_passes=False),
    )
    def k(xcol_hbm, rows_hbm, out_hbm, idx_v, rows_v, tiles_v, temp_v, g0, g1, s0, s1):
        wid = _worker_id()
        gsems = (g0, g1)
        ssems = (s0, s1)

        base = pl.multiple_of(wid * per_w, 128)
        pltpu.sync_copy(xcol_hbm.at[pl.ds(base, per_w)], idx_v)

        def fire_gather(j, b):
            off = pl.multiple_of(j * 128, 128)
            pltpu.async_copy(
                rows_hbm.at[idx_v.at[pl.ds(off, 128)]], rows_v.at[b], gsems[b],
            )

        def wait_gather(b):
            pltpu.make_async_copy(
                rows_hbm.at[pl.ds(0, 128)], rows_v.at[b], gsems[b],
            ).wait()

        def fire_stores(j, b):
            for kb in range(8):
                pltpu.async_copy(
                    tiles_v.at[b, pl.ds(kb * 8, 8), :],
                    out_hbm.at[j, kb, wid], ssems[b],
                )

        def wait_stores(b):
            pltpu.make_async_copy(
                rows_hbm.at[pl.ds(0, EMB), pl.ds(0, 128)], tiles_v.at[b],
                ssems[b],
            ).wait()

        def transpose(b):
            _transpose16(rows_v.at[b], tiles_v.at[b], temp_v,
                         128 // 16, EMB // 16)

        fire_gather(0, 0)

        def pair(p, carry):
            for b in range(2):
                j = 2 * p + b

                @pl.when(j + 1 < n_j)
                def _():
                    fire_gather(j + 1, 1 - b)

                wait_gather(b)

                @pl.when(j >= 2)
                def _():
                    wait_stores(b)

                transpose(b)
                fire_stores(j, b)
            return carry

        lax.fori_loop(0, n_j // 2, pair, 0)
        wait_stores(0)
        wait_stores(1)

    return k(xcol, rows)


def kernel(x, table):
    n_i, n_j = x.shape
    xcol = (
        x.T.astype(jnp.int32)
        .reshape(n_j, NUM_WORKERS, 128)
        .transpose(1, 0, 2)
        .reshape(-1)
    )
    tT = table.T
    tail = jnp.pad(
        lax.slice(table, (N_FULL_IB * 128, 0), (VOCAB, EMB)),
        ((0, 0), (0, 128 - EMB)),
    )
    rows = _rowize(tT, tail)
    out5d = _gather(xcol, rows)
    return out5d.transpose(2, 4, 0, 1, 3).reshape(n_i, n_j, EMB)
